# baseline (speedup 1.0000x reference)
import functools

import jax
import jax.numpy as jnp
from jax import lax
from jax.experimental import pallas as pl
from jax.experimental.pallas import tpu as pltpu
from jax.experimental.pallas import tpu_sc as plsc

N_NODES = 10000
F_IN = 128
H_OUT = 512
HID = 256
OUT_DIM = 128
PERIODS = 4
E_RAW = 160000

NC = 2
NS = 16
LANE = 16

N_PAD = 10240
NODE_ROWS_PER_TILE = N_PAD // NS

E_FULL = E_RAW + N_NODES
EDGE_ROWS = 1344
E_PAD = EDGE_ROWS * 128
ROWS_AB = EDGE_ROWS // (NC * NS)
ROWS_C = EDGE_ROWS // NS

_MESH = plsc.VectorSubcoreMesh(core_axis_name="c", subcore_axis_name="s")

_f32 = jnp.float32
_i32 = jnp.int32


def _zero_vec_ref(ref, n_lanes_groups):
    def body(i, _):
        ref[pl.ds(i * LANE, LANE)] = jnp.zeros((LANE,), _f32)
        return 0
    lax.fori_loop(0, n_lanes_groups, body, 0)


def _deg_body(dst_hbm, w_hbm, deg_out, dst_v, w_v, zero_v, acc):
    c = lax.axis_index("c")
    s = lax.axis_index("s")
    wid = s * NC + c
    row0 = wid * ROWS_AB
    pltpu.sync_copy(dst_hbm.at[pl.ds(row0, ROWS_AB)], dst_v)
    pltpu.sync_copy(w_hbm.at[pl.ds(row0, ROWS_AB)], w_v)
    _zero_vec_ref(zero_v, NODE_ROWS_PER_TILE // LANE)
    node0 = s * NODE_ROWS_PER_TILE
    pltpu.sync_copy(zero_v, acc.at[pl.ds(node0, NODE_ROWS_PER_TILE)])
    plsc.subcore_barrier()

    def batch(j, _):
        pltpu.sync_copy(w_v.at[j], acc.at[dst_v.at[j]], add=True)
        return 0
    lax.fori_loop(0, ROWS_AB, batch, 0)
    plsc.subcore_barrier()
    pltpu.sync_copy(acc.at[pl.ds(node0, NODE_ROWS_PER_TILE)],
                    deg_out.at[pl.ds(c * N_PAD + node0, NODE_ROWS_PER_TILE)])


_SC_PARAMS = pltpu.CompilerParams(use_tc_tiling_on_sc=False,
                                  needs_layout_passes=False)

_deg_call = pl.kernel(
    _deg_body,
    out_type=jax.ShapeDtypeStruct((NC * N_PAD,), _f32),
    mesh=_MESH,
    compiler_params=_SC_PARAMS,
    scratch_types=[
        pltpu.VMEM((ROWS_AB, 128), _i32),
        pltpu.VMEM((ROWS_AB, 128), _f32),
        pltpu.VMEM((NODE_ROWS_PER_TILE,), _f32),
        pltpu.VMEM_SHARED((N_PAD,), _f32),
    ],
)


def _norm_body(src_hbm, dst_hbm, w_hbm, dinv_hbm, norm_out, rs_out,
               src_v, dst_v, w_v, dinv_v, norm_v, zero_v, acc):
    c = lax.axis_index("c")
    s = lax.axis_index("s")
    wid = s * NC + c
    row0 = wid * ROWS_AB
    pltpu.sync_copy(src_hbm.at[pl.ds(row0, ROWS_AB)], src_v)
    pltpu.sync_copy(dst_hbm.at[pl.ds(row0, ROWS_AB)], dst_v)
    pltpu.sync_copy(w_hbm.at[pl.ds(row0, ROWS_AB)], w_v)
    pltpu.sync_copy(dinv_hbm, dinv_v)
    _zero_vec_ref(zero_v, NODE_ROWS_PER_TILE // LANE)
    node0 = s * NODE_ROWS_PER_TILE
    pltpu.sync_copy(zero_v, acc.at[pl.ds(node0, NODE_ROWS_PER_TILE)])
    plsc.subcore_barrier()

    def row(j, _):
        def sub(k, _):
            sl = pl.ds(k * LANE, LANE)
            sv = src_v[j, sl]
            dv = dst_v[j, sl]
            wv = w_v[j, sl]
            nv = plsc.load_gather(dinv_v, [sv]) * wv * plsc.load_gather(dinv_v, [dv])
            norm_v[j, sl] = nv
            return 0
        lax.fori_loop(0, 128 // LANE, sub, 0)
        pltpu.sync_copy(norm_v.at[j], acc.at[dst_v.at[j]], add=True)
        return 0
    lax.fori_loop(0, ROWS_AB, row, 0)
    pltpu.sync_copy(norm_v, norm_out.at[pl.ds(row0, ROWS_AB)])
    plsc.subcore_barrier()
    pltpu.sync_copy(acc.at[pl.ds(node0, NODE_ROWS_PER_TILE)],
                    rs_out.at[pl.ds(c * N_PAD + node0, NODE_ROWS_PER_TILE)])


_norm_call = pl.kernel(
    _norm_body,
    out_type=(jax.ShapeDtypeStruct((EDGE_ROWS, 128), _f32),
              jax.ShapeDtypeStruct((NC * N_PAD,), _f32)),
    mesh=_MESH,
    compiler_params=_SC_PARAMS,
    scratch_types=[
        pltpu.VMEM((ROWS_AB, 128), _i32),
        pltpu.VMEM((ROWS_AB, 128), _i32),
        pltpu.VMEM((ROWS_AB, 128), _f32),
        pltpu.VMEM((N_PAD,), _f32),
        pltpu.VMEM((ROWS_AB, 128), _f32),
        pltpu.VMEM((NODE_ROWS_PER_TILE,), _f32),
        pltpu.VMEM_SHARED((N_PAD,), _f32),
    ],
)


COLG = 8
ROWW = 16
SEG = 84
N_SEG = EDGE_ROWS // SEG
ACC_LEN = N_PAD * COLG


def _spmm_body(table_hbm, src_hbm, dst_hbm, norm_hbm, out_hbm,
               src_v, dst_v, norm_v, rows_a, rows_b, acc_ref, gs_a, gs_b):
    c = lax.axis_index("c")
    s = lax.axis_index("s")
    q16 = s // 2
    h = s % 2
    iota = jnp.arange(LANE, dtype=_i32)
    lanevec = iota - COLG * h
    mask = jnp.logical_and(iota >= COLG * h, iota < COLG * (h + 1))

    def process(buf, j):
        jvec = jnp.zeros((LANE,), _i32) + j
        for e in range(128):
            cvec = jnp.full((LANE,), e, _i32)
            nv = plsc.load_gather(norm_v, [jvec, cvec])
            dv = plsc.load_gather(dst_v, [jvec, cvec])
            addr = jnp.where(mask, dv * COLG + lanevec, 0)
            v = buf[e, :] * nv
            plsc.addupdate_scatter(acc_ref, [addr], v, mask=mask)

    for tt in range(PERIODS // NC):
        t = c + NC * tt
        idx_const = t * (N_NODES * COLG) + q16
        def zz(i, _):
            for u in range(8):
                acc_ref[pl.ds((i * 8 + u) * LANE, LANE)] = jnp.zeros((LANE,),
                                                                     _f32)
            return 0
        lax.fori_loop(0, ACC_LEN // (8 * LANE), zz, 0)

        def segment(sg, _):
            row0 = sg * SEG
            pltpu.sync_copy(src_hbm.at[pl.ds(row0, SEG)], src_v)
            pltpu.sync_copy(dst_hbm.at[pl.ds(row0, SEG)], dst_v)
            pltpu.sync_copy(norm_hbm.at[pl.ds(row0, SEG)], norm_v)
            def ix(r, _):
                for k in range(128 // LANE):
                    sl = pl.ds(k * LANE, LANE)
                    src_v[r, sl] = src_v[r, sl] * COLG + idx_const
                return 0
            lax.fori_loop(0, SEG, ix, 0)
            pltpu.async_copy(table_hbm.at[src_v.at[0]], rows_a, gs_a)

            def pair(p, _):
                j0 = 2 * p
                pltpu.make_async_copy(table_hbm.at[src_v.at[j0]],
                                      rows_a, gs_a).wait()
                pltpu.async_copy(table_hbm.at[src_v.at[j0 + 1]], rows_b, gs_b)
                process(rows_a, j0)
                pltpu.make_async_copy(table_hbm.at[src_v.at[j0 + 1]],
                                      rows_b, gs_b).wait()
                @pl.when(j0 + 2 < SEG)
                def _pre():
                    pltpu.async_copy(table_hbm.at[src_v.at[j0 + 2]],
                                     rows_a, gs_a)
                process(rows_b, j0 + 1)
                return 0
            lax.fori_loop(0, SEG // 2, pair, 0)
            return 0
        lax.fori_loop(0, N_SEG, segment, 0)
        pltpu.sync_copy(acc_ref,
                        out_hbm.at[pl.ds((t * NS + s) * ACC_LEN, ACC_LEN)])


_spmm_call = pl.kernel(
    _spmm_body,
    out_type=jax.ShapeDtypeStruct((PERIODS * NS * ACC_LEN,), _f32),
    mesh=_MESH,
    compiler_params=_SC_PARAMS,
    scratch_types=[
        pltpu.VMEM((SEG, 128), _i32),
        pltpu.VMEM((SEG, 128), _i32),
        pltpu.VMEM((SEG, 128), _f32),
        pltpu.VMEM((128, ROWW), _f32),
        pltpu.VMEM((128, ROWW), _f32),
        pltpu.VMEM((ACC_LEN,), _f32),
        pltpu.SemaphoreType.DMA,
        pltpu.SemaphoreType.DMA,
    ],
)


_BLK = 512
_GRID = N_PAD // _BLK


def _dense_body(axt_ref, rs_ref, att_ref,
                Wc_z_ref, Wl_z_ref, bc_z_ref, bl_z_ref,
                Wc_r_ref, Wl_r_ref, bc_r_ref, bl_r_ref,
                Wc_h_ref, Wl_h_ref, bc_h_ref, bl_h_ref,
                W1_ref, b1_ref, W2_ref, b2_ref,
                out_ref, hid_ref,
                wfz_s, wfr_s, wfh_s):
    @pl.when(pl.program_id(0) == 0)
    def _fold():
        wfz_s[...] = jnp.dot(Wc_z_ref[...], Wl_z_ref[:H_OUT, :],
                             preferred_element_type=_f32)
        wfr_s[...] = jnp.dot(Wc_r_ref[...], Wl_r_ref[:H_OUT, :],
                             preferred_element_type=_f32)
        wfh_s[...] = jnp.dot(Wc_h_ref[...], Wl_h_ref[:H_OUT, :],
                             preferred_element_type=_f32)

    att = att_ref[...]
    att = att - jnp.max(att, axis=1, keepdims=True)
    e = jnp.exp(att)
    probs = e / jnp.sum(e, axis=1, keepdims=True)

    rs = rs_ref[...]
    bclz = jnp.dot(bc_z_ref[...], Wl_z_ref[:H_OUT, :], preferred_element_type=_f32)
    bclr = jnp.dot(bc_r_ref[...], Wl_r_ref[:H_OUT, :], preferred_element_type=_f32)
    bclh = jnp.dot(bc_h_ref[...], Wl_h_ref[:H_OUT, :], preferred_element_type=_f32)

    H = jnp.zeros((_BLK, H_OUT), _f32)
    Hacc = jnp.zeros((_BLK, H_OUT), _f32)
    for t in range(PERIODS):
        axt = axt_ref[t]
        gz = jnp.dot(axt, wfz_s[...], preferred_element_type=_f32) + rs * bclz
        gr = jnp.dot(axt, wfr_s[...], preferred_element_type=_f32) + rs * bclr
        gh = jnp.dot(axt, wfh_s[...], preferred_element_type=_f32) + rs * bclh
        z = jax.nn.sigmoid(gz + jnp.dot(H, Wl_z_ref[H_OUT:, :],
                                        preferred_element_type=_f32) + bl_z_ref[...])
        r = jax.nn.sigmoid(gr + jnp.dot(H, Wl_r_ref[H_OUT:, :],
                                        preferred_element_type=_f32) + bl_r_ref[...])
        ht = jnp.tanh(gh + jnp.dot(H * r, Wl_h_ref[H_OUT:, :],
                                   preferred_element_type=_f32) + bl_h_ref[...])
        H = z * H + (1.0 - z) * ht
        Hacc = Hacc + probs[0, t] * H

    hid_ref[...] = Hacc
    h = jnp.maximum(Hacc, 0.0)
    h = jnp.maximum(jnp.dot(h, W1_ref[...], preferred_element_type=_f32)
                    + b1_ref[...], 0.0)
    out_ref[...] = jnp.dot(h, W2_ref[...], preferred_element_type=_f32) + b2_ref[...]


def _const_spec(shape):
    return pl.BlockSpec(shape, lambda i: tuple(0 for _ in shape))


_dense_call = pl.pallas_call(
    _dense_body,
    grid=(_GRID,),
    in_specs=[
        pl.BlockSpec((PERIODS, _BLK, F_IN), lambda i: (0, i, 0)),
        pl.BlockSpec((_BLK, 1), lambda i: (i, 0)),
        _const_spec((1, PERIODS)),
        _const_spec((F_IN, H_OUT)), _const_spec((2 * H_OUT, H_OUT)),
        _const_spec((1, H_OUT)), _const_spec((1, H_OUT)),
        _const_spec((F_IN, H_OUT)), _const_spec((2 * H_OUT, H_OUT)),
        _const_spec((1, H_OUT)), _const_spec((1, H_OUT)),
        _const_spec((F_IN, H_OUT)), _const_spec((2 * H_OUT, H_OUT)),
        _const_spec((1, H_OUT)), _const_spec((1, H_OUT)),
        _const_spec((H_OUT, HID)), _const_spec((1, HID)),
        _const_spec((HID, OUT_DIM)), _const_spec((1, OUT_DIM)),
    ],
    out_specs=[
        pl.BlockSpec((_BLK, OUT_DIM), lambda i: (i, 0)),
        pl.BlockSpec((_BLK, H_OUT), lambda i: (i, 0)),
    ],
    out_shape=[
        jax.ShapeDtypeStruct((N_PAD, OUT_DIM), _f32),
        jax.ShapeDtypeStruct((N_PAD, H_OUT), _f32),
    ],
    scratch_shapes=[
        pltpu.VMEM((F_IN, H_OUT), _f32),
        pltpu.VMEM((F_IN, H_OUT), _f32),
        pltpu.VMEM((F_IN, H_OUT), _f32),
    ],
)


def kernel(x, edge_index, edge_attr, Wc_z, bc_z, Wl_z, bl_z, Wc_r, bc_r,
           Wl_r, bl_r, Wc_h, bc_h, Wl_h, bl_h, att, W1, b1, W2, b2):
    pad_e = E_PAD - E_FULL
    loop_idx = jnp.arange(N_NODES, dtype=_i32)
    pad_idx = jnp.arange(pad_e, dtype=_i32) % N_NODES
    src_f = jnp.concatenate([edge_index[0], loop_idx, pad_idx]).reshape(EDGE_ROWS, 128)
    dst_f = jnp.concatenate([edge_index[1], loop_idx, pad_idx]).reshape(EDGE_ROWS, 128)
    w_f = jnp.concatenate([edge_attr, jnp.ones((N_NODES,), _f32),
                           jnp.zeros((pad_e,), _f32)]).reshape(EDGE_ROWS, 128)
    xT = jnp.transpose(x, (2, 0, 1)).reshape(PERIODS * N_NODES * COLG, ROWW)

    deg_part = _deg_call(dst_f, w_f)
    deg = deg_part[:N_PAD] + deg_part[N_PAD:]
    dinv = jnp.where(deg > 0, lax.rsqrt(deg), 0.0)
    norm_f, rs_part = _norm_call(src_f, dst_f, w_f, dinv)
    rs = (rs_part[:N_PAD] + rs_part[N_PAD:]).reshape(N_PAD, 1)
    ax = _spmm_call(xT, src_f, dst_f, norm_f)
    axt = (ax.reshape(PERIODS, NS, N_PAD, COLG)
             .transpose(0, 2, 1, 3)
             .reshape(PERIODS, N_PAD, F_IN))

    out, hid = _dense_call(
        axt, rs, att.reshape(1, PERIODS),
        Wc_z, Wl_z, bc_z.reshape(1, H_OUT), bl_z.reshape(1, H_OUT),
        Wc_r, Wl_r, bc_r.reshape(1, H_OUT), bl_r.reshape(1, H_OUT),
        Wc_h, Wl_h, bc_h.reshape(1, H_OUT), bl_h.reshape(1, H_OUT),
        W1, b1.reshape(1, HID), W2, b2.reshape(1, OUT_DIM))
    return out[:N_NODES], hid[:N_NODES]


# speedup vs baseline: 3.6439x; 3.6439x over previous
import functools

import jax
import jax.numpy as jnp
from jax import lax
from jax.experimental import pallas as pl
from jax.experimental.pallas import tpu as pltpu
from jax.experimental.pallas import tpu_sc as plsc

N_NODES = 10000
F_IN = 128
H_OUT = 512
HID = 256
OUT_DIM = 128
PERIODS = 4
E_RAW = 160000

NC = 2
NS = 16
LANE = 16

N_PAD = 10240
NODE_ROWS_PER_TILE = N_PAD // NS

E_FULL = E_RAW + N_NODES
EDGE_ROWS = 1344
E_PAD = EDGE_ROWS * 128
ROWS_AB = EDGE_ROWS // (NC * NS)
ROWS_C = EDGE_ROWS // NS

_MESH = plsc.VectorSubcoreMesh(core_axis_name="c", subcore_axis_name="s")

_f32 = jnp.float32
_i32 = jnp.int32


def _zero_vec_ref(ref, n_lanes_groups):
    def body(i, _):
        ref[pl.ds(i * LANE, LANE)] = jnp.zeros((LANE,), _f32)
        return 0
    lax.fori_loop(0, n_lanes_groups, body, 0)


def _deg_body(dst_hbm, w_hbm, deg_out, dst_v, w_v, zero_v, acc):
    c = lax.axis_index("c")
    s = lax.axis_index("s")
    wid = s * NC + c
    row0 = wid * ROWS_AB
    pltpu.sync_copy(dst_hbm.at[pl.ds(row0, ROWS_AB)], dst_v)
    pltpu.sync_copy(w_hbm.at[pl.ds(row0, ROWS_AB)], w_v)
    _zero_vec_ref(zero_v, NODE_ROWS_PER_TILE // LANE)
    node0 = s * NODE_ROWS_PER_TILE
    pltpu.sync_copy(zero_v, acc.at[pl.ds(node0, NODE_ROWS_PER_TILE)])
    plsc.subcore_barrier()

    def batch(j, _):
        pltpu.sync_copy(w_v.at[j], acc.at[dst_v.at[j]], add=True)
        return 0
    lax.fori_loop(0, ROWS_AB, batch, 0)
    plsc.subcore_barrier()
    pltpu.sync_copy(acc.at[pl.ds(node0, NODE_ROWS_PER_TILE)],
                    deg_out.at[pl.ds(c * N_PAD + node0, NODE_ROWS_PER_TILE)])


_SC_PARAMS = pltpu.CompilerParams(use_tc_tiling_on_sc=False,
                                  needs_layout_passes=False)

_deg_call = pl.kernel(
    _deg_body,
    out_type=jax.ShapeDtypeStruct((NC * N_PAD,), _f32),
    mesh=_MESH,
    compiler_params=_SC_PARAMS,
    scratch_types=[
        pltpu.VMEM((ROWS_AB, 128), _i32),
        pltpu.VMEM((ROWS_AB, 128), _f32),
        pltpu.VMEM((NODE_ROWS_PER_TILE,), _f32),
        pltpu.VMEM_SHARED((N_PAD,), _f32),
    ],
)


def _norm_body(src_hbm, dst_hbm, w_hbm, dinv_hbm, norm_out, rs_out,
               src_v, dst_v, w_v, dinv_v, norm_v, zero_v, acc):
    c = lax.axis_index("c")
    s = lax.axis_index("s")
    wid = s * NC + c
    row0 = wid * ROWS_AB
    pltpu.sync_copy(src_hbm.at[pl.ds(row0, ROWS_AB)], src_v)
    pltpu.sync_copy(dst_hbm.at[pl.ds(row0, ROWS_AB)], dst_v)
    pltpu.sync_copy(w_hbm.at[pl.ds(row0, ROWS_AB)], w_v)
    pltpu.sync_copy(dinv_hbm, dinv_v)
    _zero_vec_ref(zero_v, NODE_ROWS_PER_TILE // LANE)
    node0 = s * NODE_ROWS_PER_TILE
    pltpu.sync_copy(zero_v, acc.at[pl.ds(node0, NODE_ROWS_PER_TILE)])
    plsc.subcore_barrier()

    def row(j, _):
        def sub(k, _):
            sl = pl.ds(k * LANE, LANE)
            sv = src_v[j, sl]
            dv = dst_v[j, sl]
            wv = w_v[j, sl]
            nv = plsc.load_gather(dinv_v, [sv]) * wv * plsc.load_gather(dinv_v, [dv])
            norm_v[j, sl] = nv
            return 0
        lax.fori_loop(0, 128 // LANE, sub, 0)
        pltpu.sync_copy(norm_v.at[j], acc.at[dst_v.at[j]], add=True)
        return 0
    lax.fori_loop(0, ROWS_AB, row, 0)
    pltpu.sync_copy(norm_v, norm_out.at[pl.ds(row0, ROWS_AB)])
    plsc.subcore_barrier()
    pltpu.sync_copy(acc.at[pl.ds(node0, NODE_ROWS_PER_TILE)],
                    rs_out.at[pl.ds(c * N_PAD + node0, NODE_ROWS_PER_TILE)])


_norm_call = pl.kernel(
    _norm_body,
    out_type=(jax.ShapeDtypeStruct((EDGE_ROWS, 128), _f32),
              jax.ShapeDtypeStruct((NC * N_PAD,), _f32)),
    mesh=_MESH,
    compiler_params=_SC_PARAMS,
    scratch_types=[
        pltpu.VMEM((ROWS_AB, 128), _i32),
        pltpu.VMEM((ROWS_AB, 128), _i32),
        pltpu.VMEM((ROWS_AB, 128), _f32),
        pltpu.VMEM((N_PAD,), _f32),
        pltpu.VMEM((ROWS_AB, 128), _f32),
        pltpu.VMEM((NODE_ROWS_PER_TILE,), _f32),
        pltpu.VMEM_SHARED((N_PAD,), _f32),
    ],
)


HALF = 64
NBUF = 4
N_GROUPS = ROWS_C // NBUF


def _spmm_body(table_hbm, idx_hbm, dst_hbm, norm_hbm, out_hbm,
               idx_v, dst_v, norm_v, rows0, rows1, rows2, rows3, zero_v, acc,
               gs0, gs1, gs2, gs3, ss0, ss1, ss2, ss3):
    c = lax.axis_index("c")
    s = lax.axis_index("s")
    rows = (rows0, rows1, rows2, rows3)
    gsem = (gs0, gs1, gs2, gs3)
    ssem = (ss0, ss1, ss2, ss3)
    row0 = s * ROWS_C
    pltpu.sync_copy(dst_hbm.at[pl.ds(row0, ROWS_C)], dst_v)
    pltpu.sync_copy(norm_hbm.at[pl.ds(row0, ROWS_C)], norm_v)

    def zb(i, _):
        def zc(k, _):
            zero_v[i, pl.ds(k * LANE, LANE)] = jnp.zeros((LANE,), _f32)
            return 0
        lax.fori_loop(0, HALF // LANE, zc, 0)
        return 0
    lax.fori_loop(0, 64, zb, 0)

    def scale_rows(buf, j):
        def rowgrp(g, _):
            nv16 = norm_v[j, pl.ds(g * LANE, LANE)]
            for r16 in range(LANE):
                nv = nv16[r16]
                r = g * LANE + r16
                for k in range(HALF // LANE):
                    sl = pl.ds(k * LANE, LANE)
                    buf[r, sl] = buf[r, sl] * nv
            return 0
        lax.fori_loop(0, 128 // LANE, rowgrp, 0)

    node0 = s * NODE_ROWS_PER_TILE
    for tt in range(PERIODS // NC):
        t = c + NC * tt
        for h in range(2):
            chunk = t * 2 + h
            pltpu.sync_copy(idx_hbm.at[pl.ds(chunk * EDGE_ROWS + row0, ROWS_C)],
                            idx_v)
            for zi in range(NODE_ROWS_PER_TILE // 64):
                pltpu.sync_copy(zero_v, acc.at[pl.ds(node0 + zi * 64, 64)])
            plsc.subcore_barrier()

            pltpu.async_copy(table_hbm.at[idx_v.at[0]], rows[0], gsem[0])

            def group(g, _):
                for b in range(NBUF):
                    j = g * NBUF + b
                    b1 = (b + 1) % NBUF
                    pltpu.make_async_copy(table_hbm.at[idx_v.at[j]],
                                          rows[b], gsem[b]).wait()
                    scale_rows(rows[b], j)
                    pltpu.async_copy(rows[b], acc.at[dst_v.at[j]], ssem[b],
                                     add=True)
                    @pl.when(jnp.logical_and(j + 1 < ROWS_C, j + 1 >= NBUF))
                    def _drain():
                        pltpu.make_async_copy(
                            rows[b1], acc.at[dst_v.at[j]], ssem[b1]).wait()
                    @pl.when(j + 1 < ROWS_C)
                    def _fire():
                        pltpu.async_copy(table_hbm.at[idx_v.at[j + 1]],
                                         rows[b1], gsem[b1])
                return 0
            lax.fori_loop(0, N_GROUPS, group, 0)
            for b in range(NBUF):
                pltpu.make_async_copy(rows[b], acc.at[dst_v.at[0]],
                                      ssem[b]).wait()
            plsc.subcore_barrier()
            pltpu.sync_copy(
                acc.at[pl.ds(node0, NODE_ROWS_PER_TILE)],
                out_hbm.at[pl.ds(h * PERIODS * N_PAD + t * N_PAD + node0,
                                 NODE_ROWS_PER_TILE)])
            plsc.subcore_barrier()


_spmm_call = pl.kernel(
    _spmm_body,
    out_type=jax.ShapeDtypeStruct((2 * PERIODS * N_PAD, HALF), _f32),
    mesh=_MESH,
    compiler_params=_SC_PARAMS,
    scratch_types=[
        pltpu.VMEM((ROWS_C, 128), _i32),
        pltpu.VMEM((ROWS_C, 128), _i32),
        pltpu.VMEM((ROWS_C, 128), _f32),
        pltpu.VMEM((128, HALF), _f32),
        pltpu.VMEM((128, HALF), _f32),
        pltpu.VMEM((128, HALF), _f32),
        pltpu.VMEM((128, HALF), _f32),
        pltpu.VMEM((64, HALF), _f32),
        pltpu.VMEM_SHARED((N_PAD, HALF), _f32),
        pltpu.SemaphoreType.DMA,
        pltpu.SemaphoreType.DMA,
        pltpu.SemaphoreType.DMA,
        pltpu.SemaphoreType.DMA,
        pltpu.SemaphoreType.DMA,
        pltpu.SemaphoreType.DMA,
        pltpu.SemaphoreType.DMA,
        pltpu.SemaphoreType.DMA,
    ],
)


_BLK = 512
_GRID = N_PAD // _BLK


_bf16 = jnp.bfloat16


def _dense_body(axt_ref, rs_ref, att_ref,
                Wc_z_ref, Wl_z_ref, bc_z_ref, bl_z_ref,
                Wc_r_ref, Wl_r_ref, bc_r_ref, bl_r_ref,
                Wc_h_ref, Wl_h_ref, bc_h_ref, bl_h_ref,
                W1_ref, b1_ref, W2_ref, b2_ref,
                Wlzb_ref, Wlrb_ref, Wlhb_ref, W1b_ref, W2b_ref,
                out_ref, hid_ref,
                wfz_s, wfr_s, wfh_s):
    @pl.when(pl.program_id(0) == 0)
    def _fold():
        wfz_s[...] = jnp.dot(Wc_z_ref[...], Wl_z_ref[:H_OUT, :],
                             preferred_element_type=_f32).astype(_bf16)
        wfr_s[...] = jnp.dot(Wc_r_ref[...], Wl_r_ref[:H_OUT, :],
                             preferred_element_type=_f32).astype(_bf16)
        wfh_s[...] = jnp.dot(Wc_h_ref[...], Wl_h_ref[:H_OUT, :],
                             preferred_element_type=_f32).astype(_bf16)

    att = att_ref[...]
    att = att - jnp.max(att, axis=1, keepdims=True)
    e = jnp.exp(att)
    probs = e / jnp.sum(e, axis=1, keepdims=True)

    rs = rs_ref[...]
    bclz = jnp.dot(bc_z_ref[...], Wl_z_ref[:H_OUT, :], preferred_element_type=_f32)
    bclr = jnp.dot(bc_r_ref[...], Wl_r_ref[:H_OUT, :], preferred_element_type=_f32)
    bclh = jnp.dot(bc_h_ref[...], Wl_h_ref[:H_OUT, :], preferred_element_type=_f32)

    H = jnp.zeros((_BLK, H_OUT), _f32)
    Hacc = jnp.zeros((_BLK, H_OUT), _f32)
    for t in range(PERIODS):
        axtb = axt_ref[t].astype(_bf16)
        Hb = H.astype(_bf16)
        gz = jnp.dot(axtb, wfz_s[...], preferred_element_type=_f32) + rs * bclz
        gr = jnp.dot(axtb, wfr_s[...], preferred_element_type=_f32) + rs * bclr
        gh = jnp.dot(axtb, wfh_s[...], preferred_element_type=_f32) + rs * bclh
        z = jax.nn.sigmoid(gz + jnp.dot(Hb, Wlzb_ref[...],
                                        preferred_element_type=_f32) + bl_z_ref[...])
        r = jax.nn.sigmoid(gr + jnp.dot(Hb, Wlrb_ref[...],
                                        preferred_element_type=_f32) + bl_r_ref[...])
        ht = jnp.tanh(gh + jnp.dot((H * r).astype(_bf16), Wlhb_ref[...],
                                   preferred_element_type=_f32) + bl_h_ref[...])
        H = z * H + (1.0 - z) * ht
        Hacc = Hacc + probs[0, t] * H

    hid_ref[...] = Hacc
    h = jnp.maximum(Hacc, 0.0).astype(_bf16)
    h = jnp.maximum(jnp.dot(h, W1b_ref[...], preferred_element_type=_f32)
                    + b1_ref[...], 0.0).astype(_bf16)
    out_ref[...] = jnp.dot(h, W2b_ref[...], preferred_element_type=_f32) + b2_ref[...]


def _const_spec(shape):
    return pl.BlockSpec(shape, lambda i: tuple(0 for _ in shape))


_dense_call = pl.pallas_call(
    _dense_body,
    grid=(_GRID,),
    in_specs=[
        pl.BlockSpec((PERIODS, _BLK, F_IN), lambda i: (0, i, 0)),
        pl.BlockSpec((_BLK, 1), lambda i: (i, 0)),
        _const_spec((1, PERIODS)),
        _const_spec((F_IN, H_OUT)), _const_spec((2 * H_OUT, H_OUT)),
        _const_spec((1, H_OUT)), _const_spec((1, H_OUT)),
        _const_spec((F_IN, H_OUT)), _const_spec((2 * H_OUT, H_OUT)),
        _const_spec((1, H_OUT)), _const_spec((1, H_OUT)),
        _const_spec((F_IN, H_OUT)), _const_spec((2 * H_OUT, H_OUT)),
        _const_spec((1, H_OUT)), _const_spec((1, H_OUT)),
        _const_spec((H_OUT, HID)), _const_spec((1, HID)),
        _const_spec((HID, OUT_DIM)), _const_spec((1, OUT_DIM)),
        _const_spec((H_OUT, H_OUT)), _const_spec((H_OUT, H_OUT)),
        _const_spec((H_OUT, H_OUT)),
        _const_spec((H_OUT, HID)), _const_spec((HID, OUT_DIM)),
    ],
    out_specs=[
        pl.BlockSpec((_BLK, OUT_DIM), lambda i: (i, 0)),
        pl.BlockSpec((_BLK, H_OUT), lambda i: (i, 0)),
    ],
    out_shape=[
        jax.ShapeDtypeStruct((N_PAD, OUT_DIM), _f32),
        jax.ShapeDtypeStruct((N_PAD, H_OUT), _f32),
    ],
    scratch_shapes=[
        pltpu.VMEM((F_IN, H_OUT), jnp.bfloat16),
        pltpu.VMEM((F_IN, H_OUT), jnp.bfloat16),
        pltpu.VMEM((F_IN, H_OUT), jnp.bfloat16),
    ],
)


def kernel(x, edge_index, edge_attr, Wc_z, bc_z, Wl_z, bl_z, Wc_r, bc_r,
           Wl_r, bl_r, Wc_h, bc_h, Wl_h, bl_h, att, W1, b1, W2, b2):
    pad_e = E_PAD - E_FULL
    loop_idx = jnp.arange(N_NODES, dtype=_i32)
    pad_idx = jnp.arange(pad_e, dtype=_i32) % N_NODES
    src_f = jnp.concatenate([edge_index[0], loop_idx, pad_idx]).reshape(EDGE_ROWS, 128)
    dst_f = jnp.concatenate([edge_index[1], loop_idx, pad_idx]).reshape(EDGE_ROWS, 128)
    w_f = jnp.concatenate([edge_attr, jnp.ones((N_NODES,), _f32),
                           jnp.zeros((pad_e,), _f32)]).reshape(EDGE_ROWS, 128)
    toffs = (jnp.arange(PERIODS, dtype=_i32) * N_NODES)[:, None, None, None]
    hoffs = jnp.arange(2, dtype=_i32)[None, :, None, None]
    idx_all = (2 * (src_f[None, None] + toffs) + hoffs
               ).reshape(2 * PERIODS * EDGE_ROWS, 128)
    xT = jnp.transpose(x, (2, 0, 1)).reshape(2 * PERIODS * N_NODES, HALF)

    deg_part = _deg_call(dst_f, w_f)
    deg = deg_part[:N_PAD] + deg_part[N_PAD:]
    dinv = jnp.where(deg > 0, lax.rsqrt(deg), 0.0)
    norm_f, rs_part = _norm_call(src_f, dst_f, w_f, dinv)
    rs = (rs_part[:N_PAD] + rs_part[N_PAD:]).reshape(N_PAD, 1)
    ax_halves = _spmm_call(xT, idx_all, dst_f, norm_f)
    ax_halves = ax_halves.reshape(2, PERIODS, N_PAD, HALF)
    axt = jnp.concatenate([ax_halves[0], ax_halves[1]], axis=-1)

    bf16 = jnp.bfloat16
    out, hid = _dense_call(
        axt, rs, att.reshape(1, PERIODS),
        Wc_z, Wl_z, bc_z.reshape(1, H_OUT), bl_z.reshape(1, H_OUT),
        Wc_r, Wl_r, bc_r.reshape(1, H_OUT), bl_r.reshape(1, H_OUT),
        Wc_h, Wl_h, bc_h.reshape(1, H_OUT), bl_h.reshape(1, H_OUT),
        W1, b1.reshape(1, HID), W2, b2.reshape(1, OUT_DIM),
        Wl_z[H_OUT:].astype(bf16), Wl_r[H_OUT:].astype(bf16),
        Wl_h[H_OUT:].astype(bf16), W1.astype(bf16), W2.astype(bf16))
    return out[:N_NODES], hid[:N_NODES]


# speedup vs baseline: 3.9298x; 1.0784x over previous
import functools

import jax
import jax.numpy as jnp
from jax import lax
from jax.experimental import pallas as pl
from jax.experimental.pallas import tpu as pltpu
from jax.experimental.pallas import tpu_sc as plsc

N_NODES = 10000
F_IN = 128
H_OUT = 512
HID = 256
OUT_DIM = 128
PERIODS = 4
E_RAW = 160000

NC = 2
NS = 16
LANE = 16

N_PAD = 10240
NODE_ROWS_PER_TILE = N_PAD // NS

E_FULL = E_RAW + N_NODES
EDGE_ROWS = 1344
E_PAD = EDGE_ROWS * 128
ROWS_AB = EDGE_ROWS // (NC * NS)
ROWS_C = EDGE_ROWS // NS

_MESH = plsc.VectorSubcoreMesh(core_axis_name="c", subcore_axis_name="s")

_f32 = jnp.float32
_i32 = jnp.int32


def _zero_vec_ref(ref, n_lanes_groups):
    def body(i, _):
        ref[pl.ds(i * LANE, LANE)] = jnp.zeros((LANE,), _f32)
        return 0
    lax.fori_loop(0, n_lanes_groups, body, 0)


def _deg_body(dst_hbm, w_hbm, deg_out, dst_v, w_v, zero_v, acc):
    c = lax.axis_index("c")
    s = lax.axis_index("s")
    wid = s * NC + c
    row0 = wid * ROWS_AB
    pltpu.sync_copy(dst_hbm.at[pl.ds(row0, ROWS_AB)], dst_v)
    pltpu.sync_copy(w_hbm.at[pl.ds(row0, ROWS_AB)], w_v)
    _zero_vec_ref(zero_v, NODE_ROWS_PER_TILE // LANE)
    node0 = s * NODE_ROWS_PER_TILE
    pltpu.sync_copy(zero_v, acc.at[pl.ds(node0, NODE_ROWS_PER_TILE)])
    plsc.subcore_barrier()

    def batch(j, _):
        pltpu.sync_copy(w_v.at[j], acc.at[dst_v.at[j]], add=True)
        return 0
    lax.fori_loop(0, ROWS_AB, batch, 0)
    plsc.subcore_barrier()
    pltpu.sync_copy(acc.at[pl.ds(node0, NODE_ROWS_PER_TILE)],
                    deg_out.at[pl.ds(c * N_PAD + node0, NODE_ROWS_PER_TILE)])


_SC_PARAMS = pltpu.CompilerParams(use_tc_tiling_on_sc=False,
                                  needs_layout_passes=False)

_deg_call = pl.kernel(
    _deg_body,
    out_type=jax.ShapeDtypeStruct((NC * N_PAD,), _f32),
    mesh=_MESH,
    compiler_params=_SC_PARAMS,
    scratch_types=[
        pltpu.VMEM((ROWS_AB, 128), _i32),
        pltpu.VMEM((ROWS_AB, 128), _f32),
        pltpu.VMEM((NODE_ROWS_PER_TILE,), _f32),
        pltpu.VMEM_SHARED((N_PAD,), _f32),
    ],
)


def _norm_body(src_hbm, dst_hbm, w_hbm, dinv_hbm, norm_out, rs_out,
               src_v, dst_v, w_v, dinv_v, norm_v, zero_v, acc):
    c = lax.axis_index("c")
    s = lax.axis_index("s")
    wid = s * NC + c
    row0 = wid * ROWS_AB
    pltpu.sync_copy(src_hbm.at[pl.ds(row0, ROWS_AB)], src_v)
    pltpu.sync_copy(dst_hbm.at[pl.ds(row0, ROWS_AB)], dst_v)
    pltpu.sync_copy(w_hbm.at[pl.ds(row0, ROWS_AB)], w_v)
    pltpu.sync_copy(dinv_hbm, dinv_v)
    _zero_vec_ref(zero_v, NODE_ROWS_PER_TILE // LANE)
    node0 = s * NODE_ROWS_PER_TILE
    pltpu.sync_copy(zero_v, acc.at[pl.ds(node0, NODE_ROWS_PER_TILE)])
    plsc.subcore_barrier()

    def row(j, _):
        def sub(k, _):
            sl = pl.ds(k * LANE, LANE)
            sv = src_v[j, sl]
            dv = dst_v[j, sl]
            wv = w_v[j, sl]
            nv = plsc.load_gather(dinv_v, [sv]) * wv * plsc.load_gather(dinv_v, [dv])
            norm_v[j, sl] = nv
            return 0
        lax.fori_loop(0, 128 // LANE, sub, 0)
        pltpu.sync_copy(norm_v.at[j], acc.at[dst_v.at[j]], add=True)
        return 0
    lax.fori_loop(0, ROWS_AB, row, 0)
    pltpu.sync_copy(norm_v, norm_out.at[pl.ds(row0, ROWS_AB)])
    plsc.subcore_barrier()
    pltpu.sync_copy(acc.at[pl.ds(node0, NODE_ROWS_PER_TILE)],
                    rs_out.at[pl.ds(c * N_PAD + node0, NODE_ROWS_PER_TILE)])


_norm_call = pl.kernel(
    _norm_body,
    out_type=(jax.ShapeDtypeStruct((EDGE_ROWS, 128), _f32),
              jax.ShapeDtypeStruct((NC * N_PAD,), _f32)),
    mesh=_MESH,
    compiler_params=_SC_PARAMS,
    scratch_types=[
        pltpu.VMEM((ROWS_AB, 128), _i32),
        pltpu.VMEM((ROWS_AB, 128), _i32),
        pltpu.VMEM((ROWS_AB, 128), _f32),
        pltpu.VMEM((N_PAD,), _f32),
        pltpu.VMEM((ROWS_AB, 128), _f32),
        pltpu.VMEM((NODE_ROWS_PER_TILE,), _f32),
        pltpu.VMEM_SHARED((N_PAD,), _f32),
    ],
)


HALF = 64
NBUF = 4
N_GROUPS = ROWS_C // NBUF


def _spmm_body(table_hbm, idx_hbm, dst_hbm, norm_hbm, out_hbm,
               idx_v, dst_v, norm_v, rows0, rows1, rows2, rows3, zero_v, acc,
               gs0, gs1, gs2, gs3, ss0, ss1, ss2, ss3):
    c = lax.axis_index("c")
    s = lax.axis_index("s")
    rows = (rows0, rows1, rows2, rows3)
    gsem = (gs0, gs1, gs2, gs3)
    ssem = (ss0, ss1, ss2, ss3)
    row0 = s * ROWS_C
    pltpu.sync_copy(dst_hbm.at[pl.ds(row0, ROWS_C)], dst_v)
    pltpu.sync_copy(norm_hbm.at[pl.ds(row0, ROWS_C)], norm_v)

    def zb(i, _):
        def zc(k, _):
            zero_v[i, pl.ds(k * LANE, LANE)] = jnp.zeros((LANE,), _f32)
            return 0
        lax.fori_loop(0, HALF // LANE, zc, 0)
        return 0
    lax.fori_loop(0, 64, zb, 0)

    def scale_rows(buf, j):
        def rowgrp(g, _):
            nv16 = norm_v[j, pl.ds(g * LANE, LANE)]
            for r16 in range(LANE):
                nv = nv16[r16]
                r = g * LANE + r16
                for k in range(HALF // LANE):
                    sl = pl.ds(k * LANE, LANE)
                    buf[r, sl] = buf[r, sl] * nv
            return 0
        lax.fori_loop(0, 128 // LANE, rowgrp, 0)

    node0 = s * NODE_ROWS_PER_TILE
    for tt in range(PERIODS // NC):
        t = c + NC * tt
        for h in range(2):
            chunk = t * 2 + h
            pltpu.sync_copy(idx_hbm.at[pl.ds(chunk * EDGE_ROWS + row0, ROWS_C)],
                            idx_v)
            for zi in range(NODE_ROWS_PER_TILE // 64):
                pltpu.sync_copy(zero_v, acc.at[pl.ds(node0 + zi * 64, 64)])
            plsc.subcore_barrier()

            pltpu.async_copy(table_hbm.at[idx_v.at[0]], rows[0], gsem[0])

            def group(g, _):
                for b in range(NBUF):
                    j = g * NBUF + b
                    b1 = (b + 1) % NBUF
                    pltpu.make_async_copy(table_hbm.at[idx_v.at[j]],
                                          rows[b], gsem[b]).wait()
                    scale_rows(rows[b], j)
                    pltpu.async_copy(rows[b], acc.at[dst_v.at[j]], ssem[b],
                                     add=True)
                    @pl.when(jnp.logical_and(j + 1 < ROWS_C, j + 1 >= NBUF))
                    def _drain():
                        pltpu.make_async_copy(
                            rows[b1], acc.at[dst_v.at[j]], ssem[b1]).wait()
                    @pl.when(j + 1 < ROWS_C)
                    def _fire():
                        pltpu.async_copy(table_hbm.at[idx_v.at[j + 1]],
                                         rows[b1], gsem[b1])
                return 0
            lax.fori_loop(0, N_GROUPS, group, 0)
            for b in range(NBUF):
                pltpu.make_async_copy(rows[b], acc.at[dst_v.at[0]],
                                      ssem[b]).wait()
            plsc.subcore_barrier()
            pltpu.sync_copy(
                acc.at[pl.ds(node0, NODE_ROWS_PER_TILE)],
                out_hbm.at[pl.ds(t * N_PAD + node0, NODE_ROWS_PER_TILE),
                           pl.ds(h * HALF, HALF)])
            plsc.subcore_barrier()


_spmm_call = pl.kernel(
    _spmm_body,
    out_type=jax.ShapeDtypeStruct((PERIODS * N_PAD, 128), _f32),
    mesh=_MESH,
    compiler_params=_SC_PARAMS,
    scratch_types=[
        pltpu.VMEM((ROWS_C, 128), _i32),
        pltpu.VMEM((ROWS_C, 128), _i32),
        pltpu.VMEM((ROWS_C, 128), _f32),
        pltpu.VMEM((128, HALF), _f32),
        pltpu.VMEM((128, HALF), _f32),
        pltpu.VMEM((128, HALF), _f32),
        pltpu.VMEM((128, HALF), _f32),
        pltpu.VMEM((64, HALF), _f32),
        pltpu.VMEM_SHARED((N_PAD, HALF), _f32),
        pltpu.SemaphoreType.DMA,
        pltpu.SemaphoreType.DMA,
        pltpu.SemaphoreType.DMA,
        pltpu.SemaphoreType.DMA,
        pltpu.SemaphoreType.DMA,
        pltpu.SemaphoreType.DMA,
        pltpu.SemaphoreType.DMA,
        pltpu.SemaphoreType.DMA,
    ],
)


_BLK = 512
_GRID = N_PAD // _BLK


_bf16 = jnp.bfloat16


def _dense_body(axt_ref, rs_ref, att_ref,
                Wc_z_ref, Wl_z_ref, bc_z_ref, bl_z_ref,
                Wc_r_ref, Wl_r_ref, bc_r_ref, bl_r_ref,
                Wc_h_ref, Wl_h_ref, bc_h_ref, bl_h_ref,
                W1_ref, b1_ref, W2_ref, b2_ref,
                Wlzb_ref, Wlrb_ref, Wlhb_ref, W1b_ref, W2b_ref,
                out_ref, hid_ref,
                wfz_s, wfr_s, wfh_s):
    @pl.when(pl.program_id(0) == 0)
    def _fold():
        wfz_s[...] = jnp.dot(Wc_z_ref[...], Wl_z_ref[:H_OUT, :],
                             preferred_element_type=_f32).astype(_bf16)
        wfr_s[...] = jnp.dot(Wc_r_ref[...], Wl_r_ref[:H_OUT, :],
                             preferred_element_type=_f32).astype(_bf16)
        wfh_s[...] = jnp.dot(Wc_h_ref[...], Wl_h_ref[:H_OUT, :],
                             preferred_element_type=_f32).astype(_bf16)

    att = att_ref[...]
    att = att - jnp.max(att, axis=1, keepdims=True)
    e = jnp.exp(att)
    probs = e / jnp.sum(e, axis=1, keepdims=True)

    rs = rs_ref[...]
    bclz = jnp.dot(bc_z_ref[...], Wl_z_ref[:H_OUT, :], preferred_element_type=_f32)
    bclr = jnp.dot(bc_r_ref[...], Wl_r_ref[:H_OUT, :], preferred_element_type=_f32)
    bclh = jnp.dot(bc_h_ref[...], Wl_h_ref[:H_OUT, :], preferred_element_type=_f32)

    H = jnp.zeros((_BLK, H_OUT), _f32)
    Hacc = jnp.zeros((_BLK, H_OUT), _f32)
    for t in range(PERIODS):
        axtb = axt_ref[t].astype(_bf16)
        Hb = H.astype(_bf16)
        gz = jnp.dot(axtb, wfz_s[...], preferred_element_type=_f32) + rs * bclz
        gr = jnp.dot(axtb, wfr_s[...], preferred_element_type=_f32) + rs * bclr
        gh = jnp.dot(axtb, wfh_s[...], preferred_element_type=_f32) + rs * bclh
        z = jax.nn.sigmoid(gz + jnp.dot(Hb, Wlzb_ref[...],
                                        preferred_element_type=_f32) + bl_z_ref[...])
        r = jax.nn.sigmoid(gr + jnp.dot(Hb, Wlrb_ref[...],
                                        preferred_element_type=_f32) + bl_r_ref[...])
        ht = jnp.tanh(gh + jnp.dot((H * r).astype(_bf16), Wlhb_ref[...],
                                   preferred_element_type=_f32) + bl_h_ref[...])
        H = z * H + (1.0 - z) * ht
        Hacc = Hacc + probs[0, t] * H

    hid_ref[...] = Hacc
    h = jnp.maximum(Hacc, 0.0).astype(_bf16)
    h = jnp.maximum(jnp.dot(h, W1b_ref[...], preferred_element_type=_f32)
                    + b1_ref[...], 0.0).astype(_bf16)
    out_ref[...] = jnp.dot(h, W2b_ref[...], preferred_element_type=_f32) + b2_ref[...]


def _const_spec(shape):
    return pl.BlockSpec(shape, lambda i: tuple(0 for _ in shape))


_dense_call = pl.pallas_call(
    _dense_body,
    grid=(_GRID,),
    in_specs=[
        pl.BlockSpec((PERIODS, _BLK, F_IN), lambda i: (0, i, 0)),
        pl.BlockSpec((_BLK, 1), lambda i: (i, 0)),
        _const_spec((1, PERIODS)),
        _const_spec((F_IN, H_OUT)), _const_spec((2 * H_OUT, H_OUT)),
        _const_spec((1, H_OUT)), _const_spec((1, H_OUT)),
        _const_spec((F_IN, H_OUT)), _const_spec((2 * H_OUT, H_OUT)),
        _const_spec((1, H_OUT)), _const_spec((1, H_OUT)),
        _const_spec((F_IN, H_OUT)), _const_spec((2 * H_OUT, H_OUT)),
        _const_spec((1, H_OUT)), _const_spec((1, H_OUT)),
        _const_spec((H_OUT, HID)), _const_spec((1, HID)),
        _const_spec((HID, OUT_DIM)), _const_spec((1, OUT_DIM)),
        _const_spec((H_OUT, H_OUT)), _const_spec((H_OUT, H_OUT)),
        _const_spec((H_OUT, H_OUT)),
        _const_spec((H_OUT, HID)), _const_spec((HID, OUT_DIM)),
    ],
    out_specs=[
        pl.BlockSpec((_BLK, OUT_DIM), lambda i: (i, 0)),
        pl.BlockSpec((_BLK, H_OUT), lambda i: (i, 0)),
    ],
    out_shape=[
        jax.ShapeDtypeStruct((N_PAD, OUT_DIM), _f32),
        jax.ShapeDtypeStruct((N_PAD, H_OUT), _f32),
    ],
    scratch_shapes=[
        pltpu.VMEM((F_IN, H_OUT), jnp.bfloat16),
        pltpu.VMEM((F_IN, H_OUT), jnp.bfloat16),
        pltpu.VMEM((F_IN, H_OUT), jnp.bfloat16),
    ],
)


def kernel(x, edge_index, edge_attr, Wc_z, bc_z, Wl_z, bl_z, Wc_r, bc_r,
           Wl_r, bl_r, Wc_h, bc_h, Wl_h, bl_h, att, W1, b1, W2, b2):
    pad_e = E_PAD - E_FULL
    loop_idx = jnp.arange(N_NODES, dtype=_i32)
    pad_idx = jnp.arange(pad_e, dtype=_i32) % N_NODES
    src_f = jnp.concatenate([edge_index[0], loop_idx, pad_idx]).reshape(EDGE_ROWS, 128)
    dst_f = jnp.concatenate([edge_index[1], loop_idx, pad_idx]).reshape(EDGE_ROWS, 128)
    w_f = jnp.concatenate([edge_attr, jnp.ones((N_NODES,), _f32),
                           jnp.zeros((pad_e,), _f32)]).reshape(EDGE_ROWS, 128)
    toffs = (jnp.arange(PERIODS, dtype=_i32) * N_NODES)[:, None, None, None]
    hoffs = jnp.arange(2, dtype=_i32)[None, :, None, None]
    idx_all = (2 * (src_f[None, None] + toffs) + hoffs
               ).reshape(2 * PERIODS * EDGE_ROWS, 128)
    xT = jnp.transpose(x, (2, 0, 1)).reshape(2 * PERIODS * N_NODES, HALF)

    deg_part = _deg_call(dst_f, w_f)
    deg = deg_part[:N_PAD] + deg_part[N_PAD:]
    dinv = jnp.where(deg > 0, lax.rsqrt(deg), 0.0)
    norm_f, rs_part = _norm_call(src_f, dst_f, w_f, dinv)
    rs = (rs_part[:N_PAD] + rs_part[N_PAD:]).reshape(N_PAD, 1)
    axt = _spmm_call(xT, idx_all, dst_f, norm_f).reshape(PERIODS, N_PAD, F_IN)

    bf16 = jnp.bfloat16
    out, hid = _dense_call(
        axt, rs, att.reshape(1, PERIODS),
        Wc_z, Wl_z, bc_z.reshape(1, H_OUT), bl_z.reshape(1, H_OUT),
        Wc_r, Wl_r, bc_r.reshape(1, H_OUT), bl_r.reshape(1, H_OUT),
        Wc_h, Wl_h, bc_h.reshape(1, H_OUT), bl_h.reshape(1, H_OUT),
        W1, b1.reshape(1, HID), W2, b2.reshape(1, OUT_DIM),
        Wl_z[H_OUT:].astype(bf16), Wl_r[H_OUT:].astype(bf16),
        Wl_h[H_OUT:].astype(bf16), W1.astype(bf16), W2.astype(bf16))
    return out[:N_NODES], hid[:N_NODES]


# speedup vs baseline: 4.0588x; 1.0328x over previous
import functools

import jax
import jax.numpy as jnp
from jax import lax
from jax.experimental import pallas as pl
from jax.experimental.pallas import tpu as pltpu
from jax.experimental.pallas import tpu_sc as plsc

N_NODES = 10000
F_IN = 128
H_OUT = 512
HID = 256
OUT_DIM = 128
PERIODS = 4
E_RAW = 160000

NC = 2
NS = 16
LANE = 16

N_PAD = 10240
NODE_ROWS_PER_TILE = N_PAD // NS

E_FULL = E_RAW + N_NODES
EDGE_ROWS = 1344
E_PAD = EDGE_ROWS * 128
ROWS_AB = EDGE_ROWS // (NC * NS)
ROWS_C = EDGE_ROWS // NS

_MESH = plsc.VectorSubcoreMesh(core_axis_name="c", subcore_axis_name="s")

_f32 = jnp.float32
_i32 = jnp.int32


def _zero_vec_ref(ref, n_lanes_groups):
    def body(i, _):
        ref[pl.ds(i * LANE, LANE)] = jnp.zeros((LANE,), _f32)
        return 0
    lax.fori_loop(0, n_lanes_groups, body, 0)


def _deg_body(dst_hbm, w_hbm, deg_out, dst_v, w_v, zero_v, acc):
    c = lax.axis_index("c")
    s = lax.axis_index("s")
    wid = s * NC + c
    row0 = wid * ROWS_AB
    pltpu.sync_copy(dst_hbm.at[pl.ds(row0, ROWS_AB)], dst_v)
    pltpu.sync_copy(w_hbm.at[pl.ds(row0, ROWS_AB)], w_v)
    _zero_vec_ref(zero_v, NODE_ROWS_PER_TILE // LANE)
    node0 = s * NODE_ROWS_PER_TILE
    pltpu.sync_copy(zero_v, acc.at[pl.ds(node0, NODE_ROWS_PER_TILE)])
    plsc.subcore_barrier()

    def batch(j, _):
        pltpu.sync_copy(w_v.at[j], acc.at[dst_v.at[j]], add=True)
        return 0
    lax.fori_loop(0, ROWS_AB, batch, 0)
    plsc.subcore_barrier()
    pltpu.sync_copy(acc.at[pl.ds(node0, NODE_ROWS_PER_TILE)],
                    deg_out.at[pl.ds(c * N_PAD + node0, NODE_ROWS_PER_TILE)])


_SC_PARAMS = pltpu.CompilerParams(use_tc_tiling_on_sc=False,
                                  needs_layout_passes=False)

_deg_call = pl.kernel(
    _deg_body,
    out_type=jax.ShapeDtypeStruct((NC * N_PAD,), _f32),
    mesh=_MESH,
    compiler_params=_SC_PARAMS,
    scratch_types=[
        pltpu.VMEM((ROWS_AB, 128), _i32),
        pltpu.VMEM((ROWS_AB, 128), _f32),
        pltpu.VMEM((NODE_ROWS_PER_TILE,), _f32),
        pltpu.VMEM_SHARED((N_PAD,), _f32),
    ],
)


def _norm_body(src_hbm, dst_hbm, w_hbm, dinv_hbm, norm_out, rs_out,
               src_v, dst_v, w_v, dinv_v, norm_v, zero_v, acc):
    c = lax.axis_index("c")
    s = lax.axis_index("s")
    wid = s * NC + c
    row0 = wid * ROWS_AB
    pltpu.sync_copy(src_hbm.at[pl.ds(row0, ROWS_AB)], src_v)
    pltpu.sync_copy(dst_hbm.at[pl.ds(row0, ROWS_AB)], dst_v)
    pltpu.sync_copy(w_hbm.at[pl.ds(row0, ROWS_AB)], w_v)
    pltpu.sync_copy(dinv_hbm, dinv_v)
    _zero_vec_ref(zero_v, NODE_ROWS_PER_TILE // LANE)
    node0 = s * NODE_ROWS_PER_TILE
    pltpu.sync_copy(zero_v, acc.at[pl.ds(node0, NODE_ROWS_PER_TILE)])
    plsc.subcore_barrier()

    def row(j, _):
        def sub(k, _):
            sl = pl.ds(k * LANE, LANE)
            sv = src_v[j, sl]
            dv = dst_v[j, sl]
            wv = w_v[j, sl]
            nv = plsc.load_gather(dinv_v, [sv]) * wv * plsc.load_gather(dinv_v, [dv])
            norm_v[j, sl] = nv
            return 0
        lax.fori_loop(0, 128 // LANE, sub, 0)
        pltpu.sync_copy(norm_v.at[j], acc.at[dst_v.at[j]], add=True)
        return 0
    lax.fori_loop(0, ROWS_AB, row, 0)
    pltpu.sync_copy(norm_v, norm_out.at[pl.ds(row0, ROWS_AB)])
    plsc.subcore_barrier()
    pltpu.sync_copy(acc.at[pl.ds(node0, NODE_ROWS_PER_TILE)],
                    rs_out.at[pl.ds(c * N_PAD + node0, NODE_ROWS_PER_TILE)])


_norm_call = pl.kernel(
    _norm_body,
    out_type=(jax.ShapeDtypeStruct((EDGE_ROWS, 128), _f32),
              jax.ShapeDtypeStruct((NC * N_PAD,), _f32)),
    mesh=_MESH,
    compiler_params=_SC_PARAMS,
    scratch_types=[
        pltpu.VMEM((ROWS_AB, 128), _i32),
        pltpu.VMEM((ROWS_AB, 128), _i32),
        pltpu.VMEM((ROWS_AB, 128), _f32),
        pltpu.VMEM((N_PAD,), _f32),
        pltpu.VMEM((ROWS_AB, 128), _f32),
        pltpu.VMEM((NODE_ROWS_PER_TILE,), _f32),
        pltpu.VMEM_SHARED((N_PAD,), _f32),
    ],
)


HALF = 64
NBUF = 4
N_GROUPS = ROWS_C // NBUF


def _make_spmm(base):
    def body(table_hbm, idx_hbm, dst_hbm, norm_hbm, out_hbm,
             idx_v, dst_v, norm_v, rows0, rows1, rows2, rows3, zero_v, acc,
             gs0, gs1, gs2, gs3, ss0, ss1, ss2, ss3):
        c = lax.axis_index("c")
        s = lax.axis_index("s")
        rows = (rows0, rows1, rows2, rows3)
        gsem = (gs0, gs1, gs2, gs3)
        ssem = (ss0, ss1, ss2, ss3)
        row0 = s * ROWS_C
        pltpu.sync_copy(dst_hbm.at[pl.ds(row0, ROWS_C)], dst_v)
        pltpu.sync_copy(norm_hbm.at[pl.ds(row0, ROWS_C)], norm_v)

        def zb(i, _):
            def zc(k, _):
                zero_v[i, pl.ds(k * LANE, LANE)] = jnp.zeros((LANE,), _f32)
                return 0
            lax.fori_loop(0, HALF // LANE, zc, 0)
            return 0
        lax.fori_loop(0, 64, zb, 0)

        def scale_rows(buf, j):
            def rowgrp(g, _):
                nv16 = norm_v[j, pl.ds(g * LANE, LANE)]
                for r16 in range(LANE):
                    nv = nv16[r16]
                    r = g * LANE + r16
                    for k in range(HALF // LANE):
                        sl = pl.ds(k * LANE, LANE)
                        buf[r, sl] = buf[r, sl] * nv
                return 0
            lax.fori_loop(0, 128 // LANE, rowgrp, 0)

        node0 = s * NODE_ROWS_PER_TILE
        t = c + base
        for h in range(2):
            pltpu.sync_copy(
                idx_hbm.at[pl.ds((t * 2 + h) * EDGE_ROWS + row0, ROWS_C)],
                idx_v)
            for zi in range(NODE_ROWS_PER_TILE // 64):
                pltpu.sync_copy(zero_v, acc.at[pl.ds(node0 + zi * 64, 64)])
            plsc.subcore_barrier()

            pltpu.async_copy(table_hbm.at[idx_v.at[0]], rows[0], gsem[0])

            def group(g, _):
                for b in range(NBUF):
                    j = g * NBUF + b
                    b1 = (b + 1) % NBUF
                    pltpu.make_async_copy(table_hbm.at[idx_v.at[j]],
                                          rows[b], gsem[b]).wait()
                    scale_rows(rows[b], j)
                    pltpu.async_copy(rows[b], acc.at[dst_v.at[j]], ssem[b],
                                     add=True)
                    @pl.when(jnp.logical_and(j + 1 < ROWS_C, j + 1 >= NBUF))
                    def _drain():
                        pltpu.make_async_copy(
                            rows[b1], acc.at[dst_v.at[j]], ssem[b1]).wait()
                    @pl.when(j + 1 < ROWS_C)
                    def _fire():
                        pltpu.async_copy(table_hbm.at[idx_v.at[j + 1]],
                                         rows[b1], gsem[b1])
                return 0
            lax.fori_loop(0, N_GROUPS, group, 0)
            for b in range(NBUF):
                pltpu.make_async_copy(rows[b], acc.at[dst_v.at[0]],
                                      ssem[b]).wait()
            plsc.subcore_barrier()
            pltpu.sync_copy(
                acc.at[pl.ds(node0, NODE_ROWS_PER_TILE)],
                out_hbm.at[pl.ds(c * N_PAD + node0, NODE_ROWS_PER_TILE),
                           pl.ds(h * HALF, HALF)])
            plsc.subcore_barrier()

    return pl.kernel(
        body,
        out_type=jax.ShapeDtypeStruct((NC * N_PAD, 128), _f32),
        mesh=_MESH,
        compiler_params=_SC_PARAMS,
        scratch_types=[
            pltpu.VMEM((ROWS_C, 128), _i32),
            pltpu.VMEM((ROWS_C, 128), _i32),
            pltpu.VMEM((ROWS_C, 128), _f32),
            pltpu.VMEM((128, HALF), _f32),
            pltpu.VMEM((128, HALF), _f32),
            pltpu.VMEM((128, HALF), _f32),
            pltpu.VMEM((128, HALF), _f32),
            pltpu.VMEM((64, HALF), _f32),
            pltpu.VMEM_SHARED((N_PAD, HALF), _f32),
        ] + [pltpu.SemaphoreType.DMA] * 8,
    )


_spmm01 = _make_spmm(0)
_spmm23 = _make_spmm(2)


_BLK = 512
_GRID = N_PAD // _BLK


_bf16 = jnp.bfloat16


def _const_spec(shape):
    return pl.BlockSpec(shape, lambda i: tuple(0 for _ in shape))


def _make_dense(base, first):
    def body(*refs):
        axt_ref, rs_ref, att_ref = refs[0], refs[1], refs[2]
        k = 3
        if not first:
            Hin_ref, Haccin_ref = refs[3], refs[4]
            k = 5
        (Wc_z_ref, Wl_z_ref, bc_z_ref, bl_z_ref,
         Wc_r_ref, Wl_r_ref, bc_r_ref, bl_r_ref,
         Wc_h_ref, Wl_h_ref, bc_h_ref, bl_h_ref,
         W1_ref, b1_ref, W2_ref, b2_ref,
         Wlzb_ref, Wlrb_ref, Wlhb_ref, W1b_ref, W2b_ref) = refs[k:k + 21]
        o1_ref, o2_ref = refs[k + 21:k + 23]
        wfz_s, wfr_s, wfh_s = refs[k + 23:k + 26]

        @pl.when(pl.program_id(0) == 0)
        def _fold():
            wfz_s[...] = jnp.dot(Wc_z_ref[...], Wl_z_ref[:H_OUT, :],
                                 preferred_element_type=_f32).astype(_bf16)
            wfr_s[...] = jnp.dot(Wc_r_ref[...], Wl_r_ref[:H_OUT, :],
                                 preferred_element_type=_f32).astype(_bf16)
            wfh_s[...] = jnp.dot(Wc_h_ref[...], Wl_h_ref[:H_OUT, :],
                                 preferred_element_type=_f32).astype(_bf16)

        att = att_ref[...]
        att = att - jnp.max(att, axis=1, keepdims=True)
        e = jnp.exp(att)
        probs = e / jnp.sum(e, axis=1, keepdims=True)

        rs = rs_ref[...]
        bclz = jnp.dot(bc_z_ref[...], Wl_z_ref[:H_OUT, :],
                       preferred_element_type=_f32)
        bclr = jnp.dot(bc_r_ref[...], Wl_r_ref[:H_OUT, :],
                       preferred_element_type=_f32)
        bclh = jnp.dot(bc_h_ref[...], Wl_h_ref[:H_OUT, :],
                       preferred_element_type=_f32)

        if first:
            H = jnp.zeros((_BLK, H_OUT), _f32)
            Hacc = jnp.zeros((_BLK, H_OUT), _f32)
        else:
            H = Hin_ref[...]
            Hacc = Haccin_ref[...]
        for tt in range(2):
            t = base + tt
            axtb = axt_ref[tt].astype(_bf16)
            Hb = H.astype(_bf16)
            gz = jnp.dot(axtb, wfz_s[...], preferred_element_type=_f32) + rs * bclz
            gr = jnp.dot(axtb, wfr_s[...], preferred_element_type=_f32) + rs * bclr
            gh = jnp.dot(axtb, wfh_s[...], preferred_element_type=_f32) + rs * bclh
            z = jax.nn.sigmoid(gz + jnp.dot(Hb, Wlzb_ref[...],
                                            preferred_element_type=_f32)
                               + bl_z_ref[...])
            r = jax.nn.sigmoid(gr + jnp.dot(Hb, Wlrb_ref[...],
                                            preferred_element_type=_f32)
                               + bl_r_ref[...])
            ht = jnp.tanh(gh + jnp.dot((H * r).astype(_bf16), Wlhb_ref[...],
                                       preferred_element_type=_f32)
                          + bl_h_ref[...])
            H = z * H + (1.0 - z) * ht
            Hacc = Hacc + probs[0, t] * H

        if first:
            o1_ref[...] = H
            o2_ref[...] = Hacc
        else:
            o2_ref[...] = Hacc
            h = jnp.maximum(Hacc, 0.0).astype(_bf16)
            h = jnp.maximum(jnp.dot(h, W1b_ref[...], preferred_element_type=_f32)
                            + b1_ref[...], 0.0).astype(_bf16)
            o1_ref[...] = (jnp.dot(h, W2b_ref[...], preferred_element_type=_f32)
                           + b2_ref[...])

    state_specs = [] if first else [
        pl.BlockSpec((_BLK, H_OUT), lambda i: (i, 0)),
        pl.BlockSpec((_BLK, H_OUT), lambda i: (i, 0)),
    ]
    out1_shape = (N_PAD, H_OUT) if first else (N_PAD, OUT_DIM)
    return pl.pallas_call(
        body,
        grid=(_GRID,),
        in_specs=[
            pl.BlockSpec((2, _BLK, F_IN), lambda i: (0, i, 0)),
            pl.BlockSpec((_BLK, 1), lambda i: (i, 0)),
            _const_spec((1, PERIODS)),
        ] + state_specs + [
            _const_spec((F_IN, H_OUT)), _const_spec((2 * H_OUT, H_OUT)),
            _const_spec((1, H_OUT)), _const_spec((1, H_OUT)),
            _const_spec((F_IN, H_OUT)), _const_spec((2 * H_OUT, H_OUT)),
            _const_spec((1, H_OUT)), _const_spec((1, H_OUT)),
            _const_spec((F_IN, H_OUT)), _const_spec((2 * H_OUT, H_OUT)),
            _const_spec((1, H_OUT)), _const_spec((1, H_OUT)),
            _const_spec((H_OUT, HID)), _const_spec((1, HID)),
            _const_spec((HID, OUT_DIM)), _const_spec((1, OUT_DIM)),
            _const_spec((H_OUT, H_OUT)), _const_spec((H_OUT, H_OUT)),
            _const_spec((H_OUT, H_OUT)),
            _const_spec((H_OUT, HID)), _const_spec((HID, OUT_DIM)),
        ],
        out_specs=[
            pl.BlockSpec((_BLK, out1_shape[1]), lambda i: (i, 0)),
            pl.BlockSpec((_BLK, H_OUT), lambda i: (i, 0)),
        ],
        out_shape=[
            jax.ShapeDtypeStruct(out1_shape, _f32),
            jax.ShapeDtypeStruct((N_PAD, H_OUT), _f32),
        ],
        scratch_shapes=[
            pltpu.VMEM((F_IN, H_OUT), jnp.bfloat16),
            pltpu.VMEM((F_IN, H_OUT), jnp.bfloat16),
            pltpu.VMEM((F_IN, H_OUT), jnp.bfloat16),
        ],
    )


_dense01 = _make_dense(0, True)
_dense23 = _make_dense(2, False)


def kernel(x, edge_index, edge_attr, Wc_z, bc_z, Wl_z, bl_z, Wc_r, bc_r,
           Wl_r, bl_r, Wc_h, bc_h, Wl_h, bl_h, att, W1, b1, W2, b2):
    pad_e = E_PAD - E_FULL
    loop_idx = jnp.arange(N_NODES, dtype=_i32)
    pad_idx = jnp.arange(pad_e, dtype=_i32) % N_NODES
    src_f = jnp.concatenate([edge_index[0], loop_idx, pad_idx]).reshape(EDGE_ROWS, 128)
    dst_f = jnp.concatenate([edge_index[1], loop_idx, pad_idx]).reshape(EDGE_ROWS, 128)
    w_f = jnp.concatenate([edge_attr, jnp.ones((N_NODES,), _f32),
                           jnp.zeros((pad_e,), _f32)]).reshape(EDGE_ROWS, 128)
    toffs = (jnp.arange(PERIODS, dtype=_i32) * N_NODES)[:, None, None, None]
    hoffs = jnp.arange(2, dtype=_i32)[None, :, None, None]
    idx_all = (2 * (src_f[None, None] + toffs) + hoffs
               ).reshape(2 * PERIODS * EDGE_ROWS, 128)
    xT = jnp.transpose(x, (2, 0, 1)).reshape(2 * PERIODS * N_NODES, HALF)

    deg_part = _deg_call(dst_f, w_f)
    deg = deg_part[:N_PAD] + deg_part[N_PAD:]
    dinv = jnp.where(deg > 0, lax.rsqrt(deg), 0.0)
    norm_f, rs_part = _norm_call(src_f, dst_f, w_f, dinv)
    rs = (rs_part[:N_PAD] + rs_part[N_PAD:]).reshape(N_PAD, 1)
    axt01 = _spmm01(xT, idx_all, dst_f, norm_f).reshape(2, N_PAD, F_IN)
    axt23 = _spmm23(xT, idx_all, dst_f, norm_f).reshape(2, N_PAD, F_IN)

    bf16 = jnp.bfloat16
    weights = (
        Wc_z, Wl_z, bc_z.reshape(1, H_OUT), bl_z.reshape(1, H_OUT),
        Wc_r, Wl_r, bc_r.reshape(1, H_OUT), bl_r.reshape(1, H_OUT),
        Wc_h, Wl_h, bc_h.reshape(1, H_OUT), bl_h.reshape(1, H_OUT),
        W1, b1.reshape(1, HID), W2, b2.reshape(1, OUT_DIM),
        Wl_z[H_OUT:].astype(bf16), Wl_r[H_OUT:].astype(bf16),
        Wl_h[H_OUT:].astype(bf16), W1.astype(bf16), W2.astype(bf16))
    att2 = att.reshape(1, PERIODS)
    H2, Hacc2 = _dense01(axt01, rs, att2, *weights)
    out, hid = _dense23(axt23, rs, att2, H2, Hacc2, *weights)
    return out[:N_NODES], hid[:N_NODES]


# speedup vs baseline: 4.0590x; 1.0000x over previous
import jax
import jax.numpy as jnp
from jax import lax
from jax.experimental import pallas as pl
from jax.experimental.pallas import tpu as pltpu
from jax.experimental.pallas import tpu_sc as plsc

N_NODES = 10000
F_IN = 128
H_OUT = 512
HID = 256
OUT_DIM = 128
PERIODS = 4
E_RAW = 160000

NC = 2
NS = 16
LANE = 16

N_PAD = 10240
NODE_ROWS_PER_TILE = N_PAD // NS

E_FULL = E_RAW + N_NODES
EDGE_ROWS = 1344
E_PAD = EDGE_ROWS * 128
ROWS_AB = EDGE_ROWS // (NC * NS)
ROWS_C = EDGE_ROWS // NS

_MESH = plsc.VectorSubcoreMesh(core_axis_name="c", subcore_axis_name="s")

_f32 = jnp.float32
_i32 = jnp.int32


def _zero_vec_ref(ref, n_lanes_groups):
    def body(i, _):
        ref[pl.ds(i * LANE, LANE)] = jnp.zeros((LANE,), _f32)
        return 0
    lax.fori_loop(0, n_lanes_groups, body, 0)


def _deg_body(dst_hbm, w_hbm, deg_out, dst_v, w_v, zero_v, acc):
    c = lax.axis_index("c")
    s = lax.axis_index("s")
    wid = s * NC + c
    row0 = wid * ROWS_AB
    pltpu.sync_copy(dst_hbm.at[pl.ds(row0, ROWS_AB)], dst_v)
    pltpu.sync_copy(w_hbm.at[pl.ds(row0, ROWS_AB)], w_v)
    _zero_vec_ref(zero_v, NODE_ROWS_PER_TILE // LANE)
    node0 = s * NODE_ROWS_PER_TILE
    pltpu.sync_copy(zero_v, acc.at[pl.ds(node0, NODE_ROWS_PER_TILE)])
    plsc.subcore_barrier()

    def batch(j, _):
        pltpu.sync_copy(w_v.at[j], acc.at[dst_v.at[j]], add=True)
        return 0
    lax.fori_loop(0, ROWS_AB, batch, 0)
    plsc.subcore_barrier()
    pltpu.sync_copy(acc.at[pl.ds(node0, NODE_ROWS_PER_TILE)],
                    deg_out.at[pl.ds(c * N_PAD + node0, NODE_ROWS_PER_TILE)])


_SC_PARAMS = pltpu.CompilerParams(use_tc_tiling_on_sc=False,
                                  needs_layout_passes=False)

_deg_call = pl.kernel(
    _deg_body,
    out_type=jax.ShapeDtypeStruct((NC * N_PAD,), _f32),
    mesh=_MESH,
    compiler_params=_SC_PARAMS,
    scratch_types=[
        pltpu.VMEM((ROWS_AB, 128), _i32),
        pltpu.VMEM((ROWS_AB, 128), _f32),
        pltpu.VMEM((NODE_ROWS_PER_TILE,), _f32),
        pltpu.VMEM_SHARED((N_PAD,), _f32),
    ],
)


def _norm_body(src_hbm, dst_hbm, w_hbm, dinv_hbm, norm_out, rs_out,
               src_v, dst_v, w_v, dinv_v, norm_v, zero_v, acc):
    c = lax.axis_index("c")
    s = lax.axis_index("s")
    wid = s * NC + c
    row0 = wid * ROWS_AB
    pltpu.sync_copy(src_hbm.at[pl.ds(row0, ROWS_AB)], src_v)
    pltpu.sync_copy(dst_hbm.at[pl.ds(row0, ROWS_AB)], dst_v)
    pltpu.sync_copy(w_hbm.at[pl.ds(row0, ROWS_AB)], w_v)
    pltpu.sync_copy(dinv_hbm, dinv_v)
    _zero_vec_ref(zero_v, NODE_ROWS_PER_TILE // LANE)
    node0 = s * NODE_ROWS_PER_TILE
    pltpu.sync_copy(zero_v, acc.at[pl.ds(node0, NODE_ROWS_PER_TILE)])
    plsc.subcore_barrier()

    def row(j, _):
        def sub(k, _):
            sl = pl.ds(k * LANE, LANE)
            sv = src_v[j, sl]
            dv = dst_v[j, sl]
            wv = w_v[j, sl]
            nv = plsc.load_gather(dinv_v, [sv]) * wv * plsc.load_gather(dinv_v, [dv])
            norm_v[j, sl] = nv
            return 0
        lax.fori_loop(0, 128 // LANE, sub, 0)
        pltpu.sync_copy(norm_v.at[j], acc.at[dst_v.at[j]], add=True)
        return 0
    lax.fori_loop(0, ROWS_AB, row, 0)
    pltpu.sync_copy(norm_v, norm_out.at[pl.ds(row0, ROWS_AB)])
    plsc.subcore_barrier()
    pltpu.sync_copy(acc.at[pl.ds(node0, NODE_ROWS_PER_TILE)],
                    rs_out.at[pl.ds(c * N_PAD + node0, NODE_ROWS_PER_TILE)])


_norm_call = pl.kernel(
    _norm_body,
    out_type=(jax.ShapeDtypeStruct((EDGE_ROWS, 128), _f32),
              jax.ShapeDtypeStruct((NC * N_PAD,), _f32)),
    mesh=_MESH,
    compiler_params=_SC_PARAMS,
    scratch_types=[
        pltpu.VMEM((ROWS_AB, 128), _i32),
        pltpu.VMEM((ROWS_AB, 128), _i32),
        pltpu.VMEM((ROWS_AB, 128), _f32),
        pltpu.VMEM((N_PAD,), _f32),
        pltpu.VMEM((ROWS_AB, 128), _f32),
        pltpu.VMEM((NODE_ROWS_PER_TILE,), _f32),
        pltpu.VMEM_SHARED((N_PAD,), _f32),
    ],
)


HALF = 64
NBUF = 4
N_GROUPS = ROWS_C // NBUF


def _make_spmm(base):
    def body(table_hbm, idx_hbm, dst_hbm, norm_hbm, out_hbm,
             idx_v, dst_v, norm_v, rows0, rows1, rows2, rows3, zero_v, acc,
             gs0, gs1, gs2, gs3, ss0, ss1, ss2, ss3):
        c = lax.axis_index("c")
        s = lax.axis_index("s")
        rows = (rows0, rows1, rows2, rows3)
        gsem = (gs0, gs1, gs2, gs3)
        ssem = (ss0, ss1, ss2, ss3)
        row0 = s * ROWS_C
        pltpu.sync_copy(dst_hbm.at[pl.ds(row0, ROWS_C)], dst_v)
        pltpu.sync_copy(norm_hbm.at[pl.ds(row0, ROWS_C)], norm_v)

        def zb(i, _):
            def zc(k, _):
                zero_v[i, pl.ds(k * LANE, LANE)] = jnp.zeros((LANE,), _f32)
                return 0
            lax.fori_loop(0, HALF // LANE, zc, 0)
            return 0
        lax.fori_loop(0, 64, zb, 0)

        def scale_rows(buf, j):
            def rowgrp(g, _):
                nv16 = norm_v[j, pl.ds(g * LANE, LANE)]
                for r16 in range(LANE):
                    nv = nv16[r16]
                    r = g * LANE + r16
                    for k in range(HALF // LANE):
                        sl = pl.ds(k * LANE, LANE)
                        buf[r, sl] = buf[r, sl] * nv
                return 0
            lax.fori_loop(0, 128 // LANE, rowgrp, 0)

        node0 = s * NODE_ROWS_PER_TILE
        t = c + base
        for h in range(2):
            pltpu.sync_copy(
                idx_hbm.at[pl.ds((t * 2 + h) * EDGE_ROWS + row0, ROWS_C)],
                idx_v)
            for zi in range(NODE_ROWS_PER_TILE // 64):
                pltpu.sync_copy(zero_v, acc.at[pl.ds(node0 + zi * 64, 64)])
            plsc.subcore_barrier()

            pltpu.async_copy(table_hbm.at[idx_v.at[0]], rows[0], gsem[0])

            def group(g, _):
                for b in range(NBUF):
                    j = g * NBUF + b
                    b1 = (b + 1) % NBUF
                    pltpu.make_async_copy(table_hbm.at[idx_v.at[j]],
                                          rows[b], gsem[b]).wait()
                    scale_rows(rows[b], j)
                    pltpu.async_copy(rows[b], acc.at[dst_v.at[j]], ssem[b],
                                     add=True)
                    @pl.when(jnp.logical_and(j + 1 < ROWS_C, j + 1 >= NBUF))
                    def _drain():
                        pltpu.make_async_copy(
                            rows[b1], acc.at[dst_v.at[j]], ssem[b1]).wait()
                    @pl.when(j + 1 < ROWS_C)
                    def _fire():
                        pltpu.async_copy(table_hbm.at[idx_v.at[j + 1]],
                                         rows[b1], gsem[b1])
                return 0
            lax.fori_loop(0, N_GROUPS, group, 0)
            for b in range(NBUF):
                pltpu.make_async_copy(rows[b], acc.at[dst_v.at[0]],
                                      ssem[b]).wait()
            plsc.subcore_barrier()
            pltpu.sync_copy(
                acc.at[pl.ds(node0, NODE_ROWS_PER_TILE)],
                out_hbm.at[pl.ds(c * N_PAD + node0, NODE_ROWS_PER_TILE),
                           pl.ds(h * HALF, HALF)])
            plsc.subcore_barrier()

    return pl.kernel(
        body,
        out_type=jax.ShapeDtypeStruct((NC * N_PAD, 128), _f32),
        mesh=_MESH,
        compiler_params=_SC_PARAMS,
        scratch_types=[
            pltpu.VMEM((ROWS_C, 128), _i32),
            pltpu.VMEM((ROWS_C, 128), _i32),
            pltpu.VMEM((ROWS_C, 128), _f32),
            pltpu.VMEM((128, HALF), _f32),
            pltpu.VMEM((128, HALF), _f32),
            pltpu.VMEM((128, HALF), _f32),
            pltpu.VMEM((128, HALF), _f32),
            pltpu.VMEM((64, HALF), _f32),
            pltpu.VMEM_SHARED((N_PAD, HALF), _f32),
        ] + [pltpu.SemaphoreType.DMA] * 8,
    )


_spmm01 = _make_spmm(0)
_spmm23 = _make_spmm(2)


_BLK = 512
_GRID = N_PAD // _BLK


_bf16 = jnp.bfloat16


def _const_spec(shape):
    return pl.BlockSpec(shape, lambda i: tuple(0 for _ in shape))


def _make_dense(base, first):
    def body(*refs):
        axt_ref, rs_ref, att_ref = refs[0], refs[1], refs[2]
        k = 3
        if not first:
            Hin_ref, Haccin_ref = refs[3], refs[4]
            k = 5
        (Wc_z_ref, Wl_z_ref, bc_z_ref, bl_z_ref,
         Wc_r_ref, Wl_r_ref, bc_r_ref, bl_r_ref,
         Wc_h_ref, Wl_h_ref, bc_h_ref, bl_h_ref,
         W1_ref, b1_ref, W2_ref, b2_ref,
         Wlzb_ref, Wlrb_ref, Wlhb_ref, W1b_ref, W2b_ref) = refs[k:k + 21]
        o1_ref, o2_ref = refs[k + 21:k + 23]
        wfz_s, wfr_s, wfh_s = refs[k + 23:k + 26]

        @pl.when(pl.program_id(0) == 0)
        def _fold():
            wfz_s[...] = jnp.dot(Wc_z_ref[...], Wl_z_ref[:H_OUT, :],
                                 preferred_element_type=_f32).astype(_bf16)
            wfr_s[...] = jnp.dot(Wc_r_ref[...], Wl_r_ref[:H_OUT, :],
                                 preferred_element_type=_f32).astype(_bf16)
            wfh_s[...] = jnp.dot(Wc_h_ref[...], Wl_h_ref[:H_OUT, :],
                                 preferred_element_type=_f32).astype(_bf16)

        att = att_ref[...]
        att = att - jnp.max(att, axis=1, keepdims=True)
        e = jnp.exp(att)
        probs = e / jnp.sum(e, axis=1, keepdims=True)

        rs = rs_ref[...]
        bclz = jnp.dot(bc_z_ref[...], Wl_z_ref[:H_OUT, :],
                       preferred_element_type=_f32)
        bclr = jnp.dot(bc_r_ref[...], Wl_r_ref[:H_OUT, :],
                       preferred_element_type=_f32)
        bclh = jnp.dot(bc_h_ref[...], Wl_h_ref[:H_OUT, :],
                       preferred_element_type=_f32)

        if first:
            H = jnp.zeros((_BLK, H_OUT), _f32)
            Hacc = jnp.zeros((_BLK, H_OUT), _f32)
        else:
            H = Hin_ref[...]
            Hacc = Haccin_ref[...]
        for tt in range(2):
            t = base + tt
            axtb = axt_ref[tt].astype(_bf16)
            Hb = H.astype(_bf16)
            gz = jnp.dot(axtb, wfz_s[...], preferred_element_type=_f32) + rs * bclz
            gr = jnp.dot(axtb, wfr_s[...], preferred_element_type=_f32) + rs * bclr
            gh = jnp.dot(axtb, wfh_s[...], preferred_element_type=_f32) + rs * bclh
            z = jax.nn.sigmoid(gz + jnp.dot(Hb, Wlzb_ref[...],
                                            preferred_element_type=_f32)
                               + bl_z_ref[...])
            r = jax.nn.sigmoid(gr + jnp.dot(Hb, Wlrb_ref[...],
                                            preferred_element_type=_f32)
                               + bl_r_ref[...])
            ht = jnp.tanh(gh + jnp.dot((H * r).astype(_bf16), Wlhb_ref[...],
                                       preferred_element_type=_f32)
                          + bl_h_ref[...])
            H = z * H + (1.0 - z) * ht
            Hacc = Hacc + probs[0, t] * H

        if first:
            o1_ref[...] = H
            o2_ref[...] = Hacc
        else:
            o2_ref[...] = Hacc
            h = jnp.maximum(Hacc, 0.0).astype(_bf16)
            h = jnp.maximum(jnp.dot(h, W1b_ref[...], preferred_element_type=_f32)
                            + b1_ref[...], 0.0).astype(_bf16)
            o1_ref[...] = (jnp.dot(h, W2b_ref[...], preferred_element_type=_f32)
                           + b2_ref[...])

    state_specs = [] if first else [
        pl.BlockSpec((_BLK, H_OUT), lambda i: (i, 0)),
        pl.BlockSpec((_BLK, H_OUT), lambda i: (i, 0)),
    ]
    out1_shape = (N_PAD, H_OUT) if first else (N_PAD, OUT_DIM)
    return pl.pallas_call(
        body,
        grid=(_GRID,),
        in_specs=[
            pl.BlockSpec((2, _BLK, F_IN), lambda i: (0, i, 0)),
            pl.BlockSpec((_BLK, 1), lambda i: (i, 0)),
            _const_spec((1, PERIODS)),
        ] + state_specs + [
            _const_spec((F_IN, H_OUT)), _const_spec((2 * H_OUT, H_OUT)),
            _const_spec((1, H_OUT)), _const_spec((1, H_OUT)),
            _const_spec((F_IN, H_OUT)), _const_spec((2 * H_OUT, H_OUT)),
            _const_spec((1, H_OUT)), _const_spec((1, H_OUT)),
            _const_spec((F_IN, H_OUT)), _const_spec((2 * H_OUT, H_OUT)),
            _const_spec((1, H_OUT)), _const_spec((1, H_OUT)),
            _const_spec((H_OUT, HID)), _const_spec((1, HID)),
            _const_spec((HID, OUT_DIM)), _const_spec((1, OUT_DIM)),
            _const_spec((H_OUT, H_OUT)), _const_spec((H_OUT, H_OUT)),
            _const_spec((H_OUT, H_OUT)),
            _const_spec((H_OUT, HID)), _const_spec((HID, OUT_DIM)),
        ],
        out_specs=[
            pl.BlockSpec((_BLK, out1_shape[1]), lambda i: (i, 0)),
            pl.BlockSpec((_BLK, H_OUT), lambda i: (i, 0)),
        ],
        out_shape=[
            jax.ShapeDtypeStruct(out1_shape, _f32),
            jax.ShapeDtypeStruct((N_PAD, H_OUT), _f32),
        ],
        scratch_shapes=[
            pltpu.VMEM((F_IN, H_OUT), jnp.bfloat16),
            pltpu.VMEM((F_IN, H_OUT), jnp.bfloat16),
            pltpu.VMEM((F_IN, H_OUT), jnp.bfloat16),
        ],
    )


_dense01 = _make_dense(0, True)
_dense23 = _make_dense(2, False)


def kernel(x, edge_index, edge_attr, Wc_z, bc_z, Wl_z, bl_z, Wc_r, bc_r,
           Wl_r, bl_r, Wc_h, bc_h, Wl_h, bl_h, att, W1, b1, W2, b2):
    pad_e = E_PAD - E_FULL
    loop_idx = jnp.arange(N_NODES, dtype=_i32)
    pad_idx = jnp.arange(pad_e, dtype=_i32) % N_NODES
    src_f = jnp.concatenate([edge_index[0], loop_idx, pad_idx]).reshape(EDGE_ROWS, 128)
    dst_f = jnp.concatenate([edge_index[1], loop_idx, pad_idx]).reshape(EDGE_ROWS, 128)
    w_f = jnp.concatenate([edge_attr, jnp.ones((N_NODES,), _f32),
                           jnp.zeros((pad_e,), _f32)]).reshape(EDGE_ROWS, 128)
    toffs = (jnp.arange(PERIODS, dtype=_i32) * N_NODES)[:, None, None, None]
    hoffs = jnp.arange(2, dtype=_i32)[None, :, None, None]
    idx_all = (2 * (src_f[None, None] + toffs) + hoffs
               ).reshape(2 * PERIODS * EDGE_ROWS, 128)
    xT = jnp.transpose(x, (2, 0, 1)).reshape(2 * PERIODS * N_NODES, HALF)

    deg_part = _deg_call(dst_f, w_f)
    deg = deg_part[:N_PAD] + deg_part[N_PAD:]
    dinv = jnp.where(deg > 0, lax.rsqrt(deg), 0.0)
    norm_f, rs_part = _norm_call(src_f, dst_f, w_f, dinv)
    rs = (rs_part[:N_PAD] + rs_part[N_PAD:]).reshape(N_PAD, 1)
    axt01 = _spmm01(xT, idx_all, dst_f, norm_f).reshape(2, N_PAD, F_IN)
    axt23 = _spmm23(xT, idx_all, dst_f, norm_f).reshape(2, N_PAD, F_IN)

    bf16 = jnp.bfloat16
    weights = (
        Wc_z, Wl_z, bc_z.reshape(1, H_OUT), bl_z.reshape(1, H_OUT),
        Wc_r, Wl_r, bc_r.reshape(1, H_OUT), bl_r.reshape(1, H_OUT),
        Wc_h, Wl_h, bc_h.reshape(1, H_OUT), bl_h.reshape(1, H_OUT),
        W1, b1.reshape(1, HID), W2, b2.reshape(1, OUT_DIM),
        Wl_z[H_OUT:].astype(bf16), Wl_r[H_OUT:].astype(bf16),
        Wl_h[H_OUT:].astype(bf16), W1.astype(bf16), W2.astype(bf16))
    att2 = att.reshape(1, PERIODS)
    H2, Hacc2 = _dense01(axt01, rs, att2, *weights)
    out, hid = _dense23(axt23, rs, att2, H2, Hacc2, *weights)
    return out[:N_NODES], hid[:N_NODES]
